# baseline (speedup 1.0000x reference)
import functools

import jax
import jax.numpy as jnp
from jax import lax
from jax.experimental import pallas as pl
from jax.experimental.pallas import tpu as pltpu
from jax.experimental.pallas import tpu_sc as plsc

VOCAB = 1000000
DIM = 64
B = 16384
NEG = 5

NC = 2
NS = 16
NW = NC * NS

WIN = 512
SEG_WIN = 62
NWIN = NW * SEG_WIN
LS = 999424
XCOL = WIN + 128

WS_IN = 48
WS_OUT = 112

NPAD_IN = NWIN * WS_IN
NPAD_OUT = NWIN * WS_OUT

CHUNK = 512
N_OUT = B * (1 + NEG)


def _scan_body(tin, tout, tin_tail, tout_tail, wv_i, wv_o,
               im_i, im_o, xb_a, xb_b, wvb_ia, wvb_ib, wvb_oa, wvb_ob,
               rows_i, rows_o, sem):
    wid = lax.axis_index("s") * NC + lax.axis_index("c")
    w0 = wid * SEG_WIN

    phases = [
        (tin, tin_tail, wv_i, (wvb_ia, wvb_ib), rows_i, im_i, WS_IN),
        (tout, tout_tail, wv_o, (wvb_oa, wvb_ob), rows_o, im_o, WS_OUT),
    ]
    for table, tail, wv1d, wvbs, rows, im, ws in phases:

        def fire(wg, xb, wvb, table=table, tail=tail, wv1d=wv1d, ws=ws):
            sstart = jnp.minimum(wg * WIN, LS)
            pltpu.async_copy(
                wv1d.at[pl.ds(pl.multiple_of(wg * ws, 8), ws)], wvb, sem)
            pltpu.async_copy(table.at[:, pl.ds(pl.multiple_of(sstart, 128),
                                               WIN)],
                             xb.at[:, pl.ds(0, WIN)], sem)
            @pl.when(sstart == LS)
            def _():
                pltpu.sync_copy(tail, xb.at[:, pl.ds(WIN, 128)])

        def drain(xb, wvb, table=table, wv1d=wv1d, ws=ws):
            pltpu.make_async_copy(wv1d.at[pl.ds(0, ws)], wvb, sem).wait()
            pltpu.make_async_copy(table.at[:, pl.ds(0, WIN)],
                                  xb.at[:, pl.ds(0, WIN)], sem).wait()

        def proc(wg, xb, wvb, rows=rows, im=im, ws=ws):
            sstart = jnp.minimum(wg * WIN, LS)

            def grp_body(g, _):
                v16 = wvb[pl.ds(pl.multiple_of(g * 16, 16), 16)]
                col = v16 - sstart
                rowv = jax.lax.iota(jnp.int32, 16) + g * 16
                for f in range(DIM):
                    fspl = jnp.zeros((16,), jnp.int32) + f
                    val = plsc.load_gather(xb, [fspl, col])
                    plsc.store_scatter(rows, [rowv, fspl], val)
                return 0

            lax.fori_loop(0, ws // 16, grp_body, 0)
            pltpu.sync_copy(rows,
                            im.at[pl.ds(pl.multiple_of(wg * ws, 8), ws)])

        wvb_a, wvb_b = wvbs
        fire(w0, xb_a, wvb_a)

        def pair_body(j, _):
            wa = w0 + 2 * j
            drain(xb_a, wvb_a)
            fire(wa + 1, xb_b, wvb_b)
            proc(wa, xb_a, wvb_a)
            drain(xb_b, wvb_b)
            @pl.when(j < SEG_WIN // 2 - 1)
            def _():
                fire(wa + 2, xb_a, wvb_a)
            proc(wa + 1, xb_b, wvb_b)
            return 0

        lax.fori_loop(0, SEG_WIN // 2, pair_body, 0)


_scan_select = functools.partial(
    pl.kernel,
    out_type=[
        jax.ShapeDtypeStruct((NPAD_IN, 2 * DIM), jnp.float32),
        jax.ShapeDtypeStruct((NPAD_OUT, 2 * DIM), jnp.float32),
    ],
    mesh=plsc.VectorSubcoreMesh(
        core_axis_name="c", subcore_axis_name="s",
        num_cores=NC, num_subcores=NS),
    compiler_params=pltpu.CompilerParams(needs_layout_passes=False),
    scratch_types=[
        pltpu.VMEM((DIM, XCOL), jnp.float32),
        pltpu.VMEM((DIM, XCOL), jnp.float32),
        pltpu.VMEM((WS_IN,), jnp.int32),
        pltpu.VMEM((WS_IN,), jnp.int32),
        pltpu.VMEM((WS_OUT,), jnp.int32),
        pltpu.VMEM((WS_OUT,), jnp.int32),
        pltpu.VMEM((WS_IN, 2 * DIM), jnp.float32),
        pltpu.VMEM((WS_OUT, 2 * DIM), jnp.float32),
        pltpu.SemaphoreType.DMA,
    ],
)(_scan_body)


def _unsort_body(im_i, im_o, px_i, px_o, o_in, o_out, o_neg, *rest):
    idx_bufs = rest[:7]
    rows_v, sem = rest[7], rest[8]
    wid = lax.axis_index("s") * NC + lax.axis_index("c")
    pos_base = wid * (B // NW)
    neg_base = wid * (B * NEG // NW)

    pltpu.sync_copy(px_i.at[pl.ds(pos_base, CHUNK)], idx_bufs[0])
    pltpu.sync_copy(px_o.at[pl.ds(pos_base, CHUNK)], idx_bufs[1])
    for c in range(5):
        pltpu.sync_copy(px_o.at[pl.ds(B + neg_base + c * CHUNK, CHUNK)],
                        idx_bufs[2 + c])

    tasks = [(im_i, 0, o_in, pos_base), (im_o, 1, o_out, pos_base)]
    tasks += [(im_o, 2 + c, o_neg, neg_base + c * CHUNK) for c in range(5)]

    for im, row, out, base in tasks:
        pltpu.async_copy(im.at[idx_bufs[row]], rows_v, sem).wait()
        pltpu.sync_copy(rows_v.at[:, pl.ds(0, DIM)],
                        out.at[pl.ds(base, CHUNK)])


_unsort = functools.partial(
    pl.kernel,
    out_type=[
        jax.ShapeDtypeStruct((B, DIM), jnp.float32),
        jax.ShapeDtypeStruct((B, DIM), jnp.float32),
        jax.ShapeDtypeStruct((B * NEG, DIM), jnp.float32),
    ],
    mesh=plsc.VectorSubcoreMesh(
        core_axis_name="c", subcore_axis_name="s",
        num_cores=NC, num_subcores=NS),
    compiler_params=pltpu.CompilerParams(use_tc_tiling_on_sc=False),
    scratch_types=(
        [pltpu.VMEM((CHUNK,), jnp.int32) for _ in range(7)]
        + [pltpu.VMEM((CHUNK, 2 * DIM), jnp.float32),
           pltpu.SemaphoreType.DMA]
    ),
)(_unsort_body)


def _plan(words, ws):
    words = words.astype(jnp.int32)
    n = words.shape[0]
    ar = jnp.arange(n, dtype=jnp.int32)
    sw, order = lax.sort_key_val(words, ar)
    win = sw // WIN
    newrun = jnp.concatenate([jnp.ones((1,), jnp.bool_), win[1:] != win[:-1]])
    first = lax.cummax(jnp.where(newrun, ar, 0))
    slot = jnp.minimum(ar - first, ws - 1)
    sstart = jnp.minimum(jnp.arange(NWIN, dtype=jnp.int32) * WIN, LS)
    edges = jnp.arange(NWIN, dtype=jnp.int32) * WIN
    firsts = jnp.searchsorted(sw, edges, side="left").astype(jnp.int32)
    nexts = jnp.concatenate([firsts[1:], jnp.full((1,), n, jnp.int32)])
    gidx = firsts[:, None] + jnp.arange(ws, dtype=jnp.int32)[None, :]
    valid = gidx < nexts[:, None]
    wv = jnp.where(valid, sw[jnp.clip(gidx, 0, n - 1)], sstart[:, None])
    _, pidx = lax.sort_key_val(order, win * ws + slot)
    return wv.reshape(-1), pidx


def kernel(input_words, output_words, neg_words, in_table, out_table):
    wv_i, px_i = _plan(input_words, WS_IN)
    ow_all = jnp.concatenate([output_words.astype(jnp.int32),
                              neg_words.astype(jnp.int32)])
    wv_o, px_o = _plan(ow_all, WS_OUT)
    tin_tail = jnp.pad(in_table.T[:, LS + WIN:], ((0, 0), (0, DIM)))
    tout_tail = jnp.pad(out_table.T[:, LS + WIN:], ((0, 0), (0, DIM)))
    im_i, im_o = _scan_select(in_table.T, out_table.T, tin_tail, tout_tail,
                              wv_i, wv_o)
    o_in, o_out, o_neg = _unsort(im_i, im_o, px_i, px_o)
    return o_in, o_out, o_neg.reshape(B, NEG, DIM)


# speedup vs baseline: 2.8870x; 2.8870x over previous
import functools

import jax
import jax.numpy as jnp
from jax import lax
from jax.experimental import pallas as pl
from jax.experimental.pallas import tpu as pltpu
from jax.experimental.pallas import tpu_sc as plsc

VOCAB = 1000000
DIM = 64
B = 16384
NEG = 5

NC = 2
NS = 16
NW = NC * NS

CHUNK = 512
POS_PER_W = B // NW
NEG_PER_W = (B * NEG) // NW
NEG_CHUNKS = NEG_PER_W // CHUNK
N_CHUNKS = 2 + NEG_CHUNKS


def _body(iw, ow, ng, tin, tout, o_in, o_out, o_neg, *rest):
    idx_bufs = rest[:N_CHUNKS]
    rows_v, sem = rest[N_CHUNKS], rest[N_CHUNKS + 1]
    wid = lax.axis_index("s") * NC + lax.axis_index("c")
    pos_base = wid * POS_PER_W
    neg_base = wid * NEG_PER_W

    pltpu.sync_copy(iw.at[pl.ds(pos_base, CHUNK)], idx_bufs[0])
    pltpu.sync_copy(ow.at[pl.ds(pos_base, CHUNK)], idx_bufs[1])
    for c in range(NEG_CHUNKS):
        pltpu.sync_copy(ng.at[pl.ds(neg_base + c * CHUNK, CHUNK)],
                        idx_bufs[2 + c])

    tasks = [(tin, 0, o_in, pos_base), (tout, 1, o_out, pos_base)]
    tasks += [(tout, 2 + c, o_neg, neg_base + c * CHUNK)
              for c in range(NEG_CHUNKS)]

    for table, row, out, base in tasks:
        pltpu.async_copy(table.at[idx_bufs[row]], rows_v, sem).wait()
        pltpu.sync_copy(rows_v, out.at[pl.ds(base, CHUNK)])


_sc_gather = functools.partial(
    pl.kernel,
    out_type=[
        jax.ShapeDtypeStruct((B, DIM), jnp.float32),
        jax.ShapeDtypeStruct((B, DIM), jnp.float32),
        jax.ShapeDtypeStruct((B * NEG, DIM), jnp.float32),
    ],
    mesh=plsc.VectorSubcoreMesh(
        core_axis_name="c", subcore_axis_name="s",
        num_cores=NC, num_subcores=NS),
    compiler_params=pltpu.CompilerParams(use_tc_tiling_on_sc=False),
    scratch_types=(
        [pltpu.VMEM((CHUNK,), jnp.int32) for _ in range(N_CHUNKS)]
        + [pltpu.VMEM((CHUNK, DIM), jnp.float32),
           pltpu.SemaphoreType.DMA]
    ),
)(_body)


def kernel(input_words, output_words, neg_words, in_table, out_table):
    o_in, o_out, o_neg = _sc_gather(
        input_words.astype(jnp.int32), output_words.astype(jnp.int32),
        neg_words.astype(jnp.int32), in_table, out_table)
    return o_in, o_out, o_neg.reshape(B, NEG, DIM)
